# Initial kernel scaffold; baseline (speedup 1.0000x reference)
import jax
import jax.numpy as jnp
from jax.experimental import pallas as pl


def kernel(input_words, output_words, neg_samples, input_embed, output_embed):
    raise NotImplementedError("write your pallas kernel here")


# speedup vs baseline: 2.7163x; 2.7163x over previous
import jax
import jax.numpy as jnp
from jax import lax
from jax.experimental import pallas as pl
from jax.experimental.pallas import tpu as pltpu
from jax.experimental.pallas import tpu_sc as plsc

D = 128
B = 16384
S = 5
NCOL = 2 + S

NC = 2
NSUB = 16
NW = NC * NSUB
ROWS_PER_W = B // NW
CHUNK = 128
NCHUNK = ROWS_PER_W // CHUNK


def _sg_body(idx_hbm, in_tab, out_tab, out_hbm, idx_v, bufs, sems):
    wid = lax.axis_index("s") * NC + lax.axis_index("c")
    for t in range(NCHUNK):
        b0 = wid * ROWS_PER_W + t * CHUNK
        pltpu.sync_copy(idx_hbm.at[:, pl.ds(b0, CHUNK)], idx_v)
        copies = []
        for c in range(NCOL):
            tab = in_tab if c == 0 else out_tab
            copies.append(
                pltpu.async_copy(tab.at[idx_v.at[c]], bufs[c].at[:, 0], sems[c])
            )
        for c in range(NCOL):
            copies[c].wait()
            pltpu.sync_copy(bufs[c], out_hbm.at[pl.ds(b0, CHUNK), pl.ds(c, 1)])


def kernel(input_words, output_words, neg_samples, input_embed, output_embed):
    idx_all = jnp.concatenate(
        [input_words[None, :], output_words[None, :], neg_samples.T], axis=0
    ).astype(jnp.int32)
    mesh = plsc.VectorSubcoreMesh(core_axis_name="c", subcore_axis_name="s")
    f = pl.kernel(
        _sg_body,
        out_type=jax.ShapeDtypeStruct((B, NCOL, D), jnp.float32),
        mesh=mesh,
        scratch_types=[
            pltpu.VMEM((NCOL, CHUNK), jnp.int32),
            [pltpu.VMEM((CHUNK, 1, D), jnp.float32) for _ in range(NCOL)],
            [pltpu.SemaphoreType.DMA for _ in range(NCOL)],
        ],
    )
    return f(idx_all, input_embed, output_embed)


# speedup vs baseline: 2.7361x; 1.0073x over previous
import jax
import jax.numpy as jnp
from jax import lax
from jax.experimental import pallas as pl
from jax.experimental.pallas import tpu as pltpu
from jax.experimental.pallas import tpu_sc as plsc

D = 128
B = 16384
S = 5
NCOL = 2 + S

NC = 2
NSUB = 16
NW = NC * NSUB
ROWS_PER_W = B // NW
CHUNK = 64
NCHUNK = ROWS_PER_W // CHUNK
NBUF = 2


def _sg_body(idx_hbm, in_tab, out_tab, out_hbm, idx_v, bufs, g_sems, w_sems):
    wid = lax.axis_index("s") * NC + lax.axis_index("c")
    w_copies = [None] * NBUF
    for t in range(NCHUNK):
        p = t % NBUF
        b0 = wid * ROWS_PER_W + t * CHUNK
        if w_copies[p] is not None:
            for wc in w_copies[p]:
                wc.wait()
        if t % 2 == 0:
            pltpu.sync_copy(idx_hbm.at[:, pl.ds(b0, 2 * CHUNK)], idx_v)
        half = (t % 2) * CHUNK
        gcs = []
        for c in range(NCOL):
            tab = in_tab if c == 0 else out_tab
            gcs.append(
                pltpu.async_copy(
                    tab.at[idx_v.at[c, pl.ds(half, CHUNK)]],
                    bufs[p][c].at[:, 0],
                    g_sems[p],
                )
            )
        wcs = []
        for c in range(NCOL):
            gcs[c].wait()
            wcs.append(
                pltpu.async_copy(
                    bufs[p][c], out_hbm.at[pl.ds(b0, CHUNK), pl.ds(c, 1)], w_sems[p]
                )
            )
        w_copies[p] = wcs
    for p in range(NBUF):
        if w_copies[p] is not None:
            for wc in w_copies[p]:
                wc.wait()


def kernel(input_words, output_words, neg_samples, input_embed, output_embed):
    idx_all = jnp.concatenate(
        [input_words[None, :], output_words[None, :], neg_samples.T], axis=0
    ).astype(jnp.int32)
    mesh = plsc.VectorSubcoreMesh(core_axis_name="c", subcore_axis_name="s")
    f = pl.kernel(
        _sg_body,
        out_type=jax.ShapeDtypeStruct((B, NCOL, D), jnp.float32),
        mesh=mesh,
        scratch_types=[
            pltpu.VMEM((NCOL, 2 * CHUNK), jnp.int32),
            [[pltpu.VMEM((CHUNK, 1, D), jnp.float32) for _ in range(NCOL)]
             for _ in range(NBUF)],
            [pltpu.SemaphoreType.DMA for _ in range(NBUF)],
            [pltpu.SemaphoreType.DMA for _ in range(NBUF)],
        ],
    )
    return f(idx_all, input_embed, output_embed)


# speedup vs baseline: 2.7439x; 1.0029x over previous
import jax
import jax.numpy as jnp
from jax import lax
from jax.experimental import pallas as pl
from jax.experimental.pallas import tpu as pltpu
from jax.experimental.pallas import tpu_sc as plsc

D = 128
B = 16384
S = 5
NCOL = 2 + S

NC = 2
NSUB = 16
NW = NC * NSUB
ROWS_PER_W = B // NW
CHUNK = 64
NCHUNK = ROWS_PER_W // CHUNK
NBUF = 2


def _sg_body(idx_hbm, in_tab, out_tab, out_hbm, idx_v, bufs, g_sems, w_sems):
    wid = lax.axis_index("s") * NC + lax.axis_index("c")
    w_copies = [None] * NBUF
    for t in range(NCHUNK):
        p = t % NBUF
        b0 = wid * ROWS_PER_W + t * CHUNK
        if w_copies[p] is not None:
            for wc in w_copies[p]:
                wc.wait()
        if t % 2 == 0:
            pltpu.sync_copy(idx_hbm.at[:, pl.ds(b0, 2 * CHUNK)], idx_v)
        half = (t % 2) * CHUNK
        gcs = []
        for c in range(NCOL):
            tab = in_tab if c == 0 else out_tab
            gcs.append(
                pltpu.async_copy(
                    tab.at[idx_v.at[c, pl.ds(half, CHUNK)]],
                    bufs[p][c].at[:, 0],
                    g_sems[p],
                )
            )
        wcs = []
        for c in range(NCOL):
            gcs[c].wait()
            wcs.append(
                pltpu.async_copy(
                    bufs[p][c], out_hbm.at[pl.ds(b0, CHUNK), pl.ds(c, 1)], w_sems[p]
                )
            )
        w_copies[p] = wcs
    for p in range(NBUF):
        if w_copies[p] is not None:
            for wc in w_copies[p]:
                wc.wait()


def kernel(input_words, output_words, neg_samples, input_embed, output_embed):
    idx_all = jnp.concatenate(
        [input_words[None, :], output_words[None, :], neg_samples.T], axis=0
    ).astype(jnp.int32)
    mesh = plsc.VectorSubcoreMesh(core_axis_name="c", subcore_axis_name="s")
    f = pl.kernel(
        _sg_body,
        out_type=jax.ShapeDtypeStruct((B, NCOL, D), jnp.float32),
        mesh=mesh,
        compiler_params=pltpu.CompilerParams(use_tc_tiling_on_sc=True),
        scratch_types=[
            pltpu.VMEM((NCOL, 2 * CHUNK), jnp.int32),
            [[pltpu.VMEM((CHUNK, 1, D), jnp.float32) for _ in range(NCOL)]
             for _ in range(NBUF)],
            [pltpu.SemaphoreType.DMA for _ in range(NBUF)],
            [pltpu.SemaphoreType.DMA for _ in range(NBUF)],
        ],
    )
    return f(idx_all, input_embed, output_embed)


# speedup vs baseline: 4.6359x; 1.6895x over previous
import jax
import jax.numpy as jnp
from jax import lax
from jax.experimental import pallas as pl
from jax.experimental.pallas import tpu as pltpu
from jax.experimental.pallas import tpu_sc as plsc

D = 128
B = 16384
S = 5
NCOL = 2 + S

NC = 2
NSUB = 16
NW = NC * NSUB
ROWS_PER_W = B // NW
CHUNK = 128
NCHUNK = ROWS_PER_W // CHUNK
NBUF = 3


def _sg_body(idx_hbm, in_tab, out_tab, out_hbm, idx_bufs, bufs,
             i_sems, g_sems, w_sems):
    wid = lax.axis_index("s") * NC + lax.axis_index("c")
    wb = wid * ROWS_PER_W

    def load_idx(t):
        return pltpu.async_copy(
            idx_hbm.at[:, pl.ds(wb + t * CHUNK, CHUNK)],
            idx_bufs[t % 3], i_sems[t % 3],
        )

    g_cp = [None] * NBUF
    w_cp = [None] * NBUF
    i_cp = load_idx(0)
    prev = None
    for t in range(NCHUNK):
        i_cp.wait()
        if t + 1 < NCHUNK:
            i_cp = load_idx(t + 1)
        for c in range(NCOL):
            p = (t * NCOL + c) % NBUF
            if w_cp[p] is not None:
                w_cp[p].wait()
            tab = in_tab if c == 0 else out_tab
            g_cp[p] = pltpu.async_copy(
                tab.at[idx_bufs[t % 3].at[c]], bufs[p], g_sems[p]
            )
            if prev is not None:
                pp, pc, pt = prev
                g_cp[pp].wait()
                w_cp[pp] = pltpu.async_copy(
                    bufs[pp], out_hbm.at[pc, pl.ds(wb + pt * CHUNK, CHUNK)],
                    w_sems[pp],
                )
            prev = (p, c, t)
    pp, pc, pt = prev
    g_cp[pp].wait()
    w_cp[pp] = pltpu.async_copy(
        bufs[pp], out_hbm.at[pc, pl.ds(wb + pt * CHUNK, CHUNK)], w_sems[pp]
    )
    for p in range(NBUF):
        if w_cp[p] is not None:
            w_cp[p].wait()


def kernel(input_words, output_words, neg_samples, input_embed, output_embed):
    idx_all = jnp.concatenate(
        [input_words[None, :], output_words[None, :], neg_samples.T], axis=0
    ).astype(jnp.int32)
    mesh = plsc.VectorSubcoreMesh(core_axis_name="c", subcore_axis_name="s")
    f = pl.kernel(
        _sg_body,
        out_type=jax.ShapeDtypeStruct((NCOL, B, D), jnp.float32),
        mesh=mesh,
        scratch_types=[
            [pltpu.VMEM((NCOL, CHUNK), jnp.int32) for _ in range(3)],
            [pltpu.VMEM((CHUNK, D), jnp.float32) for _ in range(NBUF)],
            [pltpu.SemaphoreType.DMA for _ in range(3)],
            [pltpu.SemaphoreType.DMA for _ in range(NBUF)],
            [pltpu.SemaphoreType.DMA for _ in range(NBUF)],
        ],
    )
    out = f(idx_all, input_embed, output_embed)
    return jnp.transpose(out, (1, 0, 2))


# speedup vs baseline: 4.6557x; 1.0043x over previous
import jax
import jax.numpy as jnp
from jax import lax
from jax.experimental import pallas as pl
from jax.experimental.pallas import tpu as pltpu
from jax.experimental.pallas import tpu_sc as plsc

D = 128
B = 16384
S = 5
NCOL = 2 + S

NC = 2
NSUB = 16
NW = NC * NSUB
ROWS_PER_W = B // NW
CHUNK = 128
NCHUNK = ROWS_PER_W // CHUNK
NBUF = 4
LAG = 2


def _sg_body(idx_hbm, in_tab, out_tab, out_hbm, idx_bufs, bufs,
             i_sems, g_sems, w_sems):
    wid = lax.axis_index("s") * NC + lax.axis_index("c")
    wb = wid * ROWS_PER_W

    def load_idx(t):
        return pltpu.async_copy(
            idx_hbm.at[:, pl.ds(wb + t * CHUNK, CHUNK)],
            idx_bufs[t % 3], i_sems[t % 3],
        )

    g_cp = [None] * NBUF
    w_cp = [None] * NBUF
    i_cp = load_idx(0)
    pending = []

    def drain_one():
        pp, pc, pt = pending.pop(0)
        g_cp[pp].wait()
        w_cp[pp] = pltpu.async_copy(
            bufs[pp], out_hbm.at[pc, pl.ds(wb + pt * CHUNK, CHUNK)], w_sems[pp]
        )

    for t in range(NCHUNK):
        i_cp.wait()
        if t + 1 < NCHUNK:
            i_cp = load_idx(t + 1)
        for c in range(NCOL):
            p = (t * NCOL + c) % NBUF
            if w_cp[p] is not None:
                w_cp[p].wait()
            tab = in_tab if c == 0 else out_tab
            g_cp[p] = pltpu.async_copy(
                tab.at[idx_bufs[t % 3].at[c]], bufs[p], g_sems[p]
            )
            pending.append((p, c, t))
            if len(pending) > LAG:
                drain_one()
    while pending:
        drain_one()
    for p in range(NBUF):
        if w_cp[p] is not None:
            w_cp[p].wait()


def kernel(input_words, output_words, neg_samples, input_embed, output_embed):
    idx_all = jnp.concatenate(
        [input_words[None, :], output_words[None, :], neg_samples.T], axis=0
    ).astype(jnp.int32)
    mesh = plsc.VectorSubcoreMesh(core_axis_name="c", subcore_axis_name="s")
    f = pl.kernel(
        _sg_body,
        out_type=jax.ShapeDtypeStruct((NCOL, B, D), jnp.float32),
        mesh=mesh,
        scratch_types=[
            [pltpu.VMEM((NCOL, CHUNK), jnp.int32) for _ in range(3)],
            [pltpu.VMEM((CHUNK, D), jnp.float32) for _ in range(NBUF)],
            [pltpu.SemaphoreType.DMA for _ in range(3)],
            [pltpu.SemaphoreType.DMA for _ in range(NBUF)],
            [pltpu.SemaphoreType.DMA for _ in range(NBUF)],
        ],
    )
    out = f(idx_all, input_embed, output_embed)
    return jnp.transpose(out, (1, 0, 2))
